# Initial kernel scaffold; baseline (speedup 1.0000x reference)
import jax
import jax.numpy as jnp
from jax.experimental import pallas as pl


def kernel(feat1, grid1, feat2, grid2):
    raise NotImplementedError("write your pallas kernel here")


# speedup vs baseline: 3.4769x; 3.4769x over previous
import jax
import jax.numpy as jnp
from jax import lax
from jax.experimental import pallas as pl
from jax.experimental.pallas import tpu as pltpu
from jax.experimental.pallas import tpu_sc as plsc

N1 = 320000
N2 = 160000
C = 128
H = C // 2
G = 10000

BLK = 128
NB1 = N1 // BLK
NB2 = N2 // BLK
NTILES = 16
ZROWS = G // NTILES
ZR = 125
DR = 125
NDCH = G // DR
CNTW = 16


def _sc_body(feat1_hbm, grid1_hbm, feat2_hbm, grid2_hbm,
             sum1_hbm, sum2_hbm, cnt1_hbm, cnt2_hbm,
             half_v, idx_v, ones_v, zbuf, zbuf16, dbuf, dbuf16,
             ssum, scnt):
    c = lax.axis_index("c")
    s = lax.axis_index("s")

    def fill_zbuf(r, _):
        for j in range(H // 16):
            zbuf[r, pl.ds(j * 16, 16)] = jnp.zeros((16,), jnp.float32)
        zbuf16[r, :] = jnp.zeros((16,), jnp.float32)
        return 0

    lax.fori_loop(0, ZR, fill_zbuf, 0)

    def fill_ones(r, _):
        ones_v[r, :] = jnp.ones((16,), jnp.float32)
        return 0

    lax.fori_loop(0, BLK, fill_ones, 0)

    def zero_accums(zero_cnt):
        r0 = s * ZROWS
        for k in range(ZROWS // ZR):
            pltpu.sync_copy(zbuf, ssum.at[pl.ds(r0 + k * ZR, ZR)])
            if zero_cnt:
                pltpu.sync_copy(zbuf16, scnt.at[pl.ds(r0 + k * ZR, ZR)])

    zero_accums(True)
    plsc.subcore_barrier()

    def do_side(feat_hbm, grid_hbm, nb, hoff, with_cnt):
        base_blk = nb // NTILES
        extra = nb % NTILES
        my_nblk = base_blk + jnp.where(s < extra, 1, 0)
        my_start = s * base_blk + jnp.minimum(s, extra)

        def body(i, _):
            row0 = (my_start + i) * BLK
            pltpu.sync_copy(grid_hbm.at[pl.ds(row0, BLK)], idx_v)
            pltpu.sync_copy(feat_hbm.at[pl.ds(row0, BLK), pl.ds(hoff, H)],
                            half_v)
            pltpu.sync_copy(half_v, ssum.at[idx_v], add=True)
            if with_cnt:
                pltpu.sync_copy(ones_v, scnt.at[idx_v], add=True)
            return 0

        lax.fori_loop(0, my_nblk, body, 0)

    def drain(sum_hbm, cnt_hbm, hoff):
        base_ch = NDCH // NTILES
        my_ch0 = s * base_ch

        def body(i, _):
            rr = (my_ch0 + i) * DR
            pltpu.sync_copy(ssum.at[pl.ds(rr, DR)], dbuf)
            pltpu.sync_copy(dbuf, sum_hbm.at[pl.ds(rr, DR), pl.ds(hoff, H)])
            if cnt_hbm is not None:
                pltpu.sync_copy(scnt.at[pl.ds(rr, DR)], dbuf16)
                pltpu.sync_copy(dbuf16, cnt_hbm.at[pl.ds(rr, DR)])
            return 0

        lax.fori_loop(0, base_ch, body, 0)

    @pl.when(c == 0)
    def _():
        do_side(feat1_hbm, grid1_hbm, NB1, 0, True)

    @pl.when(c == 1)
    def _():
        do_side(feat1_hbm, grid1_hbm, NB1, H, False)

    plsc.subcore_barrier()

    @pl.when(c == 0)
    def _():
        drain(sum1_hbm, cnt1_hbm, 0)

    @pl.when(c == 1)
    def _():
        drain(sum1_hbm, None, H)

    plsc.subcore_barrier()

    zero_accums(False)
    plsc.subcore_barrier()

    @pl.when(c == 0)
    def _():
        do_side(feat2_hbm, grid2_hbm, NB2, 0, False)

    @pl.when(c == 1)
    def _():
        do_side(feat2_hbm, grid2_hbm, NB2, H, True)

    plsc.subcore_barrier()

    @pl.when(c == 0)
    def _():
        drain(sum2_hbm, None, 0)

    @pl.when(c == 1)
    def _():
        drain(sum2_hbm, cnt2_hbm, H)


@jax.jit
def _aggregate(feat1, grid1, feat2, grid2):
    mesh = plsc.VectorSubcoreMesh(core_axis_name="c", subcore_axis_name="s")
    f32 = jnp.float32
    return pl.kernel(
        _sc_body,
        out_type=[
            jax.ShapeDtypeStruct((G, C), f32),
            jax.ShapeDtypeStruct((G, C), f32),
            jax.ShapeDtypeStruct((G, CNTW), f32),
            jax.ShapeDtypeStruct((G, CNTW), f32),
        ],
        mesh=mesh,
        compiler_params=pltpu.CompilerParams(use_tc_tiling_on_sc=False),
        scratch_types=[
            pltpu.VMEM((BLK, H), f32),
            pltpu.VMEM((BLK,), jnp.int32),
            pltpu.VMEM((BLK, CNTW), f32),
            pltpu.VMEM((ZR, H), f32),
            pltpu.VMEM((ZR, CNTW), f32),
            pltpu.VMEM((DR, H), f32),
            pltpu.VMEM((DR, CNTW), f32),
            pltpu.VMEM_SHARED((G, H), f32),
            pltpu.VMEM_SHARED((G, CNTW), f32),
        ],
    )(feat1, grid1, feat2, grid2)


def _loss_body(s1_ref, s2_ref, c1_ref, c2_ref, out_ref):
    cnt1 = c1_ref[:, 0:1]
    cnt2 = c2_ref[:, 0:1]
    mask = jnp.logical_and(cnt1 > 0.0, cnt2 > 0.0).astype(jnp.float32)
    d = s1_ref[...] / jnp.maximum(cnt1, 1.0) - s2_ref[...] / jnp.maximum(
        cnt2, 1.0)
    total = jnp.sum(d * d * mask)
    n = jnp.sum(mask)
    loss = jnp.where(n > 0.0, total / (n * jnp.float32(C)), jnp.float32(0.0))
    out_ref[...] = jnp.broadcast_to(loss, (1, 1))


@jax.jit
def kernel(feat1, grid1, feat2, grid2):
    grid1 = grid1.astype(jnp.int32)
    grid2 = grid2.astype(jnp.int32)
    sum1, sum2, cnt1, cnt2 = _aggregate(feat1, grid1, feat2, grid2)
    out = pl.pallas_call(
        _loss_body,
        out_shape=jax.ShapeDtypeStruct((1, 1), jnp.float32),
    )(sum1, sum2, cnt1, cnt2)
    return out[0, 0]


# speedup vs baseline: 7.5467x; 2.1705x over previous
import jax
import jax.numpy as jnp
from jax import lax
from jax.experimental import pallas as pl
from jax.experimental.pallas import tpu as pltpu
from jax.experimental.pallas import tpu_sc as plsc

N1 = 320000
N2 = 160000
C = 128
H = C // 2
G = 10000

BLK = 128
NB1 = N1 // BLK
NB2 = N2 // BLK
NTILES = 16
ZROWS = G // NTILES
ZR = 125
DR = 125
NDCH = G // DR
CNTW = 16


def _sc_body(feat1_hbm, grid1_hbm, feat2_hbm, grid2_hbm,
             sum1_hbm, sum2_hbm, cnt1_hbm, cnt2_hbm,
             half0, half1, idx0, idx1, ones_v, zbuf, zbuf16, dbuf, dbuf16,
             semf0, semf1, semi0, semi1, ssum, scnt):
    c = lax.axis_index("c")
    s = lax.axis_index("s")

    def fill_zbuf(r, _):
        for j in range(H // 16):
            zbuf[r, pl.ds(j * 16, 16)] = jnp.zeros((16,), jnp.float32)
        zbuf16[r, :] = jnp.zeros((16,), jnp.float32)
        return 0

    lax.fori_loop(0, ZR, fill_zbuf, 0)

    def fill_ones(r, _):
        ones_v[r, :] = jnp.ones((16,), jnp.float32)
        return 0

    lax.fori_loop(0, BLK, fill_ones, 0)

    def zero_accums(zero_cnt):
        r0 = s * ZROWS
        for k in range(ZROWS // ZR):
            pltpu.sync_copy(zbuf, ssum.at[pl.ds(r0 + k * ZR, ZR)])
            if zero_cnt:
                pltpu.sync_copy(zbuf16, scnt.at[pl.ds(r0 + k * ZR, ZR)])

    zero_accums(True)
    plsc.subcore_barrier()

    halves = (half0, half1)
    idxs = (idx0, idx1)
    fsems = (semf0, semf1)
    isems = (semi0, semi1)

    def do_side(feat_hbm, grid_hbm, nb, hoff, with_cnt):
        base_blk = nb // NTILES
        extra = nb % NTILES
        my_nblk = base_blk + jnp.where(s < extra, 1, 0)
        my_start = s * base_blk + jnp.minimum(s, extra)

        def load(j, slot):
            row0 = (my_start + j) * BLK
            pltpu.make_async_copy(grid_hbm.at[pl.ds(row0, BLK)],
                                  idxs[slot], isems[slot]).start()
            pltpu.make_async_copy(
                feat_hbm.at[pl.ds(row0, BLK), pl.ds(hoff, H)],
                halves[slot], fsems[slot]).start()

        def wait_load(slot):
            pltpu.make_async_copy(grid_hbm.at[pl.ds(0, BLK)],
                                  idxs[slot], isems[slot]).wait()
            pltpu.make_async_copy(
                feat_hbm.at[pl.ds(0, BLK), pl.ds(0, H)],
                halves[slot], fsems[slot]).wait()

        load(0, 0)

        def pair(p, _):
            for b in range(2):
                j = p * 2 + b

                @pl.when(j < my_nblk)
                def _():
                    @pl.when(j + 1 < my_nblk)
                    def _():
                        load(j + 1, 1 - b)

                    wait_load(b)
                    pltpu.sync_copy(halves[b], ssum.at[idxs[b]], add=True)
                    if with_cnt:
                        pltpu.sync_copy(ones_v, scnt.at[idxs[b]], add=True)

            return 0

        lax.fori_loop(0, (my_nblk + 1) // 2, pair, 0)

    def drain(sum_hbm, cnt_hbm, hoff):
        base_ch = NDCH // NTILES
        my_ch0 = s * base_ch

        def body(i, _):
            rr = (my_ch0 + i) * DR
            pltpu.sync_copy(ssum.at[pl.ds(rr, DR)], dbuf)
            pltpu.sync_copy(dbuf, sum_hbm.at[pl.ds(rr, DR), pl.ds(hoff, H)])
            if cnt_hbm is not None:
                pltpu.sync_copy(scnt.at[pl.ds(rr, DR)], dbuf16)
                pltpu.sync_copy(dbuf16, cnt_hbm.at[pl.ds(rr, DR)])
            return 0

        lax.fori_loop(0, base_ch, body, 0)

    @pl.when(c == 0)
    def _():
        do_side(feat1_hbm, grid1_hbm, NB1, 0, True)

    @pl.when(c == 1)
    def _():
        do_side(feat1_hbm, grid1_hbm, NB1, H, False)

    plsc.subcore_barrier()

    @pl.when(c == 0)
    def _():
        drain(sum1_hbm, cnt1_hbm, 0)

    @pl.when(c == 1)
    def _():
        drain(sum1_hbm, None, H)

    plsc.subcore_barrier()

    zero_accums(False)
    plsc.subcore_barrier()

    @pl.when(c == 0)
    def _():
        do_side(feat2_hbm, grid2_hbm, NB2, 0, False)

    @pl.when(c == 1)
    def _():
        do_side(feat2_hbm, grid2_hbm, NB2, H, True)

    plsc.subcore_barrier()

    @pl.when(c == 0)
    def _():
        drain(sum2_hbm, None, 0)

    @pl.when(c == 1)
    def _():
        drain(sum2_hbm, cnt2_hbm, H)


@jax.jit
def _aggregate(feat1, grid1, feat2, grid2):
    mesh = plsc.VectorSubcoreMesh(core_axis_name="c", subcore_axis_name="s")
    f32 = jnp.float32
    return pl.kernel(
        _sc_body,
        out_type=[
            jax.ShapeDtypeStruct((G, C), f32),
            jax.ShapeDtypeStruct((G, C), f32),
            jax.ShapeDtypeStruct((G, CNTW), f32),
            jax.ShapeDtypeStruct((G, CNTW), f32),
        ],
        mesh=mesh,
        compiler_params=pltpu.CompilerParams(use_tc_tiling_on_sc=False),
        scratch_types=[
            pltpu.VMEM((BLK, H), f32),
            pltpu.VMEM((BLK, H), f32),
            pltpu.VMEM((BLK,), jnp.int32),
            pltpu.VMEM((BLK,), jnp.int32),
            pltpu.VMEM((BLK, CNTW), f32),
            pltpu.VMEM((ZR, H), f32),
            pltpu.VMEM((ZR, CNTW), f32),
            pltpu.VMEM((DR, H), f32),
            pltpu.VMEM((DR, CNTW), f32),
            pltpu.SemaphoreType.DMA,
            pltpu.SemaphoreType.DMA,
            pltpu.SemaphoreType.DMA,
            pltpu.SemaphoreType.DMA,
            pltpu.VMEM_SHARED((G, H), f32),
            pltpu.VMEM_SHARED((G, CNTW), f32),
        ],
    )(feat1, grid1, feat2, grid2)


def _loss_body(s1_ref, s2_ref, c1_ref, c2_ref, out_ref):
    cnt1 = c1_ref[:, 0:1]
    cnt2 = c2_ref[:, 0:1]
    mask = jnp.logical_and(cnt1 > 0.0, cnt2 > 0.0).astype(jnp.float32)
    d = s1_ref[...] / jnp.maximum(cnt1, 1.0) - s2_ref[...] / jnp.maximum(
        cnt2, 1.0)
    total = jnp.sum(d * d * mask)
    n = jnp.sum(mask)
    loss = jnp.where(n > 0.0, total / (n * jnp.float32(C)), jnp.float32(0.0))
    out_ref[...] = jnp.broadcast_to(loss, (1, 1))


@jax.jit
def kernel(feat1, grid1, feat2, grid2):
    grid1 = grid1.astype(jnp.int32)
    grid2 = grid2.astype(jnp.int32)
    sum1, sum2, cnt1, cnt2 = _aggregate(feat1, grid1, feat2, grid2)
    out = pl.pallas_call(
        _loss_body,
        out_shape=jax.ShapeDtypeStruct((1, 1), jnp.float32),
    )(sum1, sum2, cnt1, cnt2)
    return out[0, 0]


# speedup vs baseline: 8.0701x; 1.0694x over previous
import jax
import jax.numpy as jnp
from jax import lax
from jax.experimental import pallas as pl
from jax.experimental.pallas import tpu as pltpu
from jax.experimental.pallas import tpu_sc as plsc

N1 = 320000
N2 = 160000
C = 128
H = C // 2
G = 10000

BLK = 128
NB1 = N1 // BLK
NB2 = N2 // BLK
NTILES = 16
ZROWS = G // NTILES
ZR = 125
DR = 125
NDCH = G // DR
CNTW = 16


def _sc_body(feat1_hbm, grid1_hbm, feat2_hbm, grid2_hbm,
             sum1_hbm, sum2_hbm, cnt1_hbm, cnt2_hbm,
             half0, half1, idx0, idx1, cnt_part, zbuf, dbuf,
             semf0, semf1, semi0, semi1, ssum):
    c = lax.axis_index("c")
    s = lax.axis_index("s")

    def fill_zbuf(r, _):
        for j in range(H // 16):
            zbuf[r, pl.ds(j * 16, 16)] = jnp.zeros((16,), jnp.float32)
        return 0

    lax.fori_loop(0, ZR, fill_zbuf, 0)

    def fill_zcnt(r, _):
        cnt_part[pl.ds(r * 16, 16)] = jnp.zeros((16,), jnp.float32)
        return 0

    lax.fori_loop(0, G // 16, fill_zcnt, 0)

    def zero_accums():
        r0 = s * ZROWS
        for k in range(ZROWS // ZR):
            pltpu.sync_copy(zbuf, ssum.at[pl.ds(r0 + k * ZR, ZR)])

    zero_accums()
    plsc.subcore_barrier()

    halves = (half0, half1)
    idxs = (idx0, idx1)
    fsems = (semf0, semf1)
    isems = (semi0, semi1)

    def do_side(feat_hbm, grid_hbm, nb, hoff, with_cnt):
        base_blk = nb // NTILES
        extra = nb % NTILES
        my_nblk = base_blk + jnp.where(s < extra, 1, 0)
        my_start = s * base_blk + jnp.minimum(s, extra)

        def load(j, slot):
            row0 = (my_start + j) * BLK
            pltpu.make_async_copy(grid_hbm.at[pl.ds(row0, BLK)],
                                  idxs[slot], isems[slot]).start()
            pltpu.make_async_copy(
                feat_hbm.at[pl.ds(row0, BLK), pl.ds(hoff, H)],
                halves[slot], fsems[slot]).start()

        def wait_load(slot):
            pltpu.make_async_copy(grid_hbm.at[pl.ds(0, BLK)],
                                  idxs[slot], isems[slot]).wait()
            pltpu.make_async_copy(
                feat_hbm.at[pl.ds(0, BLK), pl.ds(0, H)],
                halves[slot], fsems[slot]).wait()

        load(0, 0)

        def pair(p, _):
            for b in range(2):
                j = p * 2 + b

                @pl.when(j < my_nblk)
                def _():
                    @pl.when(j + 1 < my_nblk)
                    def _():
                        load(j + 1, 1 - b)

                    wait_load(b)
                    pltpu.sync_copy(halves[b], ssum.at[idxs[b]], add=True)
                    if with_cnt:
                        ones16 = jnp.ones((16,), jnp.float32)
                        for k in range(BLK // 16):
                            idxv = idxs[b][pl.ds(k * 16, 16)]
                            plsc.addupdate_scatter(cnt_part, [idxv], ones16)

            return 0

        lax.fori_loop(0, (my_nblk + 1) // 2, pair, 0)

    def drain(sum_hbm, cnt_hbm, hoff):
        base_ch = NDCH // NTILES
        my_ch0 = s * base_ch

        def body(i, _):
            rr = (my_ch0 + i) * DR
            pltpu.sync_copy(ssum.at[pl.ds(rr, DR)], dbuf)
            pltpu.sync_copy(dbuf, sum_hbm.at[pl.ds(rr, DR), pl.ds(hoff, H)])
            return 0

        lax.fori_loop(0, base_ch, body, 0)
        if cnt_hbm is not None:
            pltpu.sync_copy(cnt_part, cnt_hbm.at[s])

    @pl.when(c == 0)
    def _():
        do_side(feat1_hbm, grid1_hbm, NB1, 0, True)

    @pl.when(c == 1)
    def _():
        do_side(feat1_hbm, grid1_hbm, NB1, H, False)

    plsc.subcore_barrier()

    @pl.when(c == 0)
    def _():
        drain(sum1_hbm, cnt1_hbm, 0)

    @pl.when(c == 1)
    def _():
        drain(sum1_hbm, None, H)

    plsc.subcore_barrier()

    zero_accums()
    plsc.subcore_barrier()

    @pl.when(c == 0)
    def _():
        do_side(feat2_hbm, grid2_hbm, NB2, 0, False)

    @pl.when(c == 1)
    def _():
        do_side(feat2_hbm, grid2_hbm, NB2, H, True)

    plsc.subcore_barrier()

    @pl.when(c == 0)
    def _():
        drain(sum2_hbm, None, 0)

    @pl.when(c == 1)
    def _():
        drain(sum2_hbm, cnt2_hbm, H)


@jax.jit
def _aggregate(feat1, grid1, feat2, grid2):
    mesh = plsc.VectorSubcoreMesh(core_axis_name="c", subcore_axis_name="s")
    f32 = jnp.float32
    return pl.kernel(
        _sc_body,
        out_type=[
            jax.ShapeDtypeStruct((G, C), f32),
            jax.ShapeDtypeStruct((G, C), f32),
            jax.ShapeDtypeStruct((NTILES, G), f32),
            jax.ShapeDtypeStruct((NTILES, G), f32),
        ],
        mesh=mesh,
        compiler_params=pltpu.CompilerParams(use_tc_tiling_on_sc=False,
                                             needs_layout_passes=False),
        scratch_types=[
            pltpu.VMEM((BLK, H), f32),
            pltpu.VMEM((BLK, H), f32),
            pltpu.VMEM((BLK,), jnp.int32),
            pltpu.VMEM((BLK,), jnp.int32),
            pltpu.VMEM((G,), f32),
            pltpu.VMEM((ZR, H), f32),
            pltpu.VMEM((DR, H), f32),
            pltpu.SemaphoreType.DMA,
            pltpu.SemaphoreType.DMA,
            pltpu.SemaphoreType.DMA,
            pltpu.SemaphoreType.DMA,
            pltpu.VMEM_SHARED((G, H), f32),
        ],
    )(feat1, grid1, feat2, grid2)


def _loss_body(s1_ref, s2_ref, c1_ref, c2_ref, out_ref):
    cnt1 = jnp.sum(c1_ref[...], axis=0)[:, None]
    cnt2 = jnp.sum(c2_ref[...], axis=0)[:, None]
    mask = jnp.logical_and(cnt1 > 0.0, cnt2 > 0.0).astype(jnp.float32)
    d = s1_ref[...] / jnp.maximum(cnt1, 1.0) - s2_ref[...] / jnp.maximum(
        cnt2, 1.0)
    total = jnp.sum(d * d * mask)
    n = jnp.sum(mask)
    loss = jnp.where(n > 0.0, total / (n * jnp.float32(C)), jnp.float32(0.0))
    out_ref[...] = jnp.broadcast_to(loss, (1, 1))


@jax.jit
def kernel(feat1, grid1, feat2, grid2):
    grid1 = grid1.astype(jnp.int32)
    grid2 = grid2.astype(jnp.int32)
    sum1, sum2, cnt1, cnt2 = _aggregate(feat1, grid1, feat2, grid2)
    out = pl.pallas_call(
        _loss_body,
        out_shape=jax.ShapeDtypeStruct((1, 1), jnp.float32),
    )(sum1, sum2, cnt1, cnt2)
    return out[0, 0]


# speedup vs baseline: 9.7388x; 1.2068x over previous
import jax
import jax.numpy as jnp
from jax import lax
from jax.experimental import pallas as pl
from jax.experimental.pallas import tpu as pltpu
from jax.experimental.pallas import tpu_sc as plsc

N1 = 320000
N2 = 160000
C = 128
H = C // 2
G = 10000

BLK = 128
NB1 = N1 // BLK
NB2 = N2 // BLK
NTILES = 16
ZROWS = G // NTILES
ZR = 125
DR = 125
NDCH = G // DR
CNTW = 16


def _sc_body(feat1_hbm, grid1_hbm, feat2_hbm, grid2_hbm,
             sum1_hbm, sum2_hbm, cnt1_hbm, cnt2_hbm,
             half0, half1, half2, half3, idx0, idx1, idx2, idx3,
             cnt_part, zbuf, dbuf,
             semf0, semf1, semf2, semf3, semi0, semi1, semi2, semi3,
             sems0, sems1, sems2, sems3, ssum):
    c = lax.axis_index("c")
    s = lax.axis_index("s")

    def fill_zbuf(r, _):
        for j in range(H // 16):
            zbuf[r, pl.ds(j * 16, 16)] = jnp.zeros((16,), jnp.float32)
        return 0

    lax.fori_loop(0, ZR, fill_zbuf, 0)

    def fill_zcnt(r, _):
        cnt_part[pl.ds(r * 16, 16)] = jnp.zeros((16,), jnp.float32)
        return 0

    lax.fori_loop(0, G // 16, fill_zcnt, 0)

    def zero_accums():
        r0 = s * ZROWS
        for k in range(ZROWS // ZR):
            pltpu.sync_copy(zbuf, ssum.at[pl.ds(r0 + k * ZR, ZR)])

    zero_accums()
    plsc.subcore_barrier()

    NS = 4
    halves = (half0, half1, half2, half3)
    idxs = (idx0, idx1, idx2, idx3)
    fsems = (semf0, semf1, semf2, semf3)
    isems = (semi0, semi1, semi2, semi3)
    ssems = (sems0, sems1, sems2, sems3)

    def do_side(feat_hbm, grid_hbm, nb, hoff, with_cnt):
        base_blk = nb // NTILES
        extra = nb % NTILES
        my_nblk = base_blk + jnp.where(s < extra, 1, 0)
        my_start = s * base_blk + jnp.minimum(s, extra)

        def load(j, slot):
            row0 = (my_start + j) * BLK
            pltpu.make_async_copy(grid_hbm.at[pl.ds(row0, BLK)],
                                  idxs[slot], isems[slot]).start()
            pltpu.make_async_copy(
                feat_hbm.at[pl.ds(row0, BLK), pl.ds(hoff, H)],
                halves[slot], fsems[slot]).start()

        def wait_load(slot):
            pltpu.make_async_copy(grid_hbm.at[pl.ds(0, BLK)],
                                  idxs[slot], isems[slot]).wait()
            pltpu.make_async_copy(
                feat_hbm.at[pl.ds(0, BLK), pl.ds(0, H)],
                halves[slot], fsems[slot]).wait()

        def wait_scatter(slot):
            pltpu.make_async_copy(halves[slot], ssum.at[idxs[slot]],
                                  ssems[slot]).wait()

        for j0 in range(NS - 1):
            load(j0, j0)

        def quad(q, _):
            for b in range(NS):
                j = q * NS + b

                @pl.when(j < my_nblk)
                def _():
                    wait_load(b)
                    pltpu.async_copy(halves[b], ssum.at[idxs[b]],
                                     ssems[b], add=True)
                    if with_cnt:
                        ones16 = jnp.ones((16,), jnp.float32)
                        for k in range(BLK // 16):
                            idxv = idxs[b][pl.ds(k * 16, 16)]
                            plsc.addupdate_scatter(cnt_part, [idxv], ones16)

                    @pl.when(j >= 1)
                    def _():
                        wait_scatter((b + NS - 1) % NS)

                    @pl.when(j + NS - 1 < my_nblk)
                    def _():
                        load(j + NS - 1, (b + NS - 1) % NS)

            return 0

        lax.fori_loop(0, (my_nblk + NS - 1) // NS, quad, 0)

        last = (my_nblk - 1) % NS
        for b in range(NS):
            @pl.when(last == b)
            def _():
                wait_scatter(b)

    def drain(sum_hbm, cnt_hbm, hoff):
        base_ch = NDCH // NTILES
        my_ch0 = s * base_ch

        def body(i, _):
            rr = (my_ch0 + i) * DR
            pltpu.sync_copy(ssum.at[pl.ds(rr, DR)], dbuf)
            pltpu.sync_copy(dbuf, sum_hbm.at[pl.ds(rr, DR), pl.ds(hoff, H)])
            return 0

        lax.fori_loop(0, base_ch, body, 0)
        if cnt_hbm is not None:
            pltpu.sync_copy(cnt_part, cnt_hbm.at[s])

    @pl.when(c == 0)
    def _():
        do_side(feat1_hbm, grid1_hbm, NB1, 0, True)

    @pl.when(c == 1)
    def _():
        do_side(feat1_hbm, grid1_hbm, NB1, H, False)

    plsc.subcore_barrier()

    @pl.when(c == 0)
    def _():
        drain(sum1_hbm, cnt1_hbm, 0)

    @pl.when(c == 1)
    def _():
        drain(sum1_hbm, None, H)

    plsc.subcore_barrier()

    zero_accums()
    plsc.subcore_barrier()

    @pl.when(c == 0)
    def _():
        do_side(feat2_hbm, grid2_hbm, NB2, 0, False)

    @pl.when(c == 1)
    def _():
        do_side(feat2_hbm, grid2_hbm, NB2, H, True)

    plsc.subcore_barrier()

    @pl.when(c == 0)
    def _():
        drain(sum2_hbm, None, 0)

    @pl.when(c == 1)
    def _():
        drain(sum2_hbm, cnt2_hbm, H)


@jax.jit
def _aggregate(feat1, grid1, feat2, grid2):
    mesh = plsc.VectorSubcoreMesh(core_axis_name="c", subcore_axis_name="s")
    f32 = jnp.float32
    return pl.kernel(
        _sc_body,
        out_type=[
            jax.ShapeDtypeStruct((G, C), f32),
            jax.ShapeDtypeStruct((G, C), f32),
            jax.ShapeDtypeStruct((NTILES, G), f32),
            jax.ShapeDtypeStruct((NTILES, G), f32),
        ],
        mesh=mesh,
        compiler_params=pltpu.CompilerParams(use_tc_tiling_on_sc=False,
                                             needs_layout_passes=False),
        scratch_types=[
            pltpu.VMEM((BLK, H), f32),
            pltpu.VMEM((BLK, H), f32),
            pltpu.VMEM((BLK, H), f32),
            pltpu.VMEM((BLK, H), f32),
            pltpu.VMEM((BLK,), jnp.int32),
            pltpu.VMEM((BLK,), jnp.int32),
            pltpu.VMEM((BLK,), jnp.int32),
            pltpu.VMEM((BLK,), jnp.int32),
            pltpu.VMEM((G,), f32),
            pltpu.VMEM((ZR, H), f32),
            pltpu.VMEM((DR, H), f32),
            pltpu.SemaphoreType.DMA,
            pltpu.SemaphoreType.DMA,
            pltpu.SemaphoreType.DMA,
            pltpu.SemaphoreType.DMA,
            pltpu.SemaphoreType.DMA,
            pltpu.SemaphoreType.DMA,
            pltpu.SemaphoreType.DMA,
            pltpu.SemaphoreType.DMA,
            pltpu.SemaphoreType.DMA,
            pltpu.SemaphoreType.DMA,
            pltpu.SemaphoreType.DMA,
            pltpu.SemaphoreType.DMA,
            pltpu.VMEM_SHARED((G, H), f32),
        ],
    )(feat1, grid1, feat2, grid2)


def _loss_body(s1_ref, s2_ref, c1_ref, c2_ref, out_ref):
    cnt1 = jnp.sum(c1_ref[...], axis=0)[:, None]
    cnt2 = jnp.sum(c2_ref[...], axis=0)[:, None]
    mask = jnp.logical_and(cnt1 > 0.0, cnt2 > 0.0).astype(jnp.float32)
    d = s1_ref[...] / jnp.maximum(cnt1, 1.0) - s2_ref[...] / jnp.maximum(
        cnt2, 1.0)
    total = jnp.sum(d * d * mask)
    n = jnp.sum(mask)
    loss = jnp.where(n > 0.0, total / (n * jnp.float32(C)), jnp.float32(0.0))
    out_ref[...] = jnp.broadcast_to(loss, (1, 1))


@jax.jit
def kernel(feat1, grid1, feat2, grid2):
    grid1 = grid1.astype(jnp.int32)
    grid2 = grid2.astype(jnp.int32)
    sum1, sum2, cnt1, cnt2 = _aggregate(feat1, grid1, feat2, grid2)
    out = pl.pallas_call(
        _loss_body,
        out_shape=jax.ShapeDtypeStruct((1, 1), jnp.float32),
    )(sum1, sum2, cnt1, cnt2)
    return out[0, 0]


# speedup vs baseline: 9.9010x; 1.0167x over previous
import jax
import jax.numpy as jnp
from jax import lax
from jax.experimental import pallas as pl
from jax.experimental.pallas import tpu as pltpu
from jax.experimental.pallas import tpu_sc as plsc

N1 = 320000
N2 = 160000
C = 128
H = C // 2
G = 10000

BLK = 128
NB1 = N1 // BLK
NB2 = N2 // BLK
NTILES = 16
ZROWS = G // NTILES
ZR = 125
DR = 125
NDCH = G // DR
NCH = NDCH // NTILES


def _sc_body(feat1_hbm, grid1_hbm, feat2_hbm, grid2_hbm, z2_hbm, z1_hbm,
             sum1_hbm, sum2_hbm, cnt1_hbm, cnt2_hbm,
             half0, half1, half2, half3, idx0, idx1, idx2, idx3,
             cnt_part, zbuf,
             semf0, semf1, semf2, semf3, semi0, semi1, semi2, semi3,
             sems0, sems1, sems2, sems3, ssum):
    c = lax.axis_index("c")
    s = lax.axis_index("s")

    NS = 4
    halves = (half0, half1, half2, half3)
    idxs = (idx0, idx1, idx2, idx3)
    fsems = (semf0, semf1, semf2, semf3)
    isems = (semi0, semi1, semi2, semi3)
    ssems = (sems0, sems1, sems2, sems3)

    def my_block_range(nb):
        base_blk = nb // NTILES
        extra = nb % NTILES
        my_nblk = base_blk + jnp.where(s < extra, 1, 0)
        my_start = s * base_blk + jnp.minimum(s, extra)
        return my_start, my_nblk

    def load_block(feat_hbm, grid_hbm, hoff, my_start, j, slot):
        row0 = (my_start + j) * BLK
        pltpu.make_async_copy(grid_hbm.at[pl.ds(row0, BLK)],
                              idxs[slot], isems[slot]).start()
        pltpu.make_async_copy(
            feat_hbm.at[pl.ds(row0, BLK), pl.ds(hoff, H)],
            halves[slot], fsems[slot]).start()

    def side_start(feat_hbm, grid_hbm, nb, hoff):
        my_start, _ = my_block_range(nb)
        for j0 in range(NS - 1):
            load_block(feat_hbm, grid_hbm, hoff, my_start, j0, j0)

    @pl.when(c == 0)
    def _():
        side_start(feat1_hbm, grid1_hbm, NB1, 0)

    @pl.when(c == 1)
    def _():
        side_start(feat1_hbm, grid1_hbm, NB1, H)

    pltpu.sync_copy(z2_hbm, zbuf)
    pltpu.sync_copy(z1_hbm, cnt_part)

    r0 = s * ZROWS
    for k in range(ZROWS // ZR):
        pltpu.sync_copy(zbuf, ssum.at[pl.ds(r0 + k * ZR, ZR)])
    plsc.subcore_barrier()

    def side_loop(feat_hbm, grid_hbm, nb, hoff, with_cnt):
        my_start, my_nblk = my_block_range(nb)

        def wait_load(slot):
            pltpu.make_async_copy(grid_hbm.at[pl.ds(0, BLK)],
                                  idxs[slot], isems[slot]).wait()
            pltpu.make_async_copy(
                feat_hbm.at[pl.ds(0, BLK), pl.ds(0, H)],
                halves[slot], fsems[slot]).wait()

        def wait_scatter(slot):
            pltpu.make_async_copy(halves[slot], ssum.at[idxs[slot]],
                                  ssems[slot]).wait()

        def quad(q, _):
            for b in range(NS):
                j = q * NS + b

                @pl.when(j < my_nblk)
                def _():
                    wait_load(b)
                    pltpu.async_copy(halves[b], ssum.at[idxs[b]],
                                     ssems[b], add=True)
                    if with_cnt:
                        ones16 = jnp.ones((16,), jnp.float32)
                        for k in range(BLK // 16):
                            idxv = idxs[b][pl.ds(k * 16, 16)]
                            plsc.addupdate_scatter(cnt_part, [idxv], ones16)

                    @pl.when(j >= 1)
                    def _():
                        wait_scatter((b + NS - 1) % NS)

                    @pl.when(j + NS - 1 < my_nblk)
                    def _():
                        load_block(feat_hbm, grid_hbm, hoff, my_start,
                                   j + NS - 1, (b + NS - 1) % NS)

            return 0

        lax.fori_loop(0, (my_nblk + NS - 1) // NS, quad, 0)

        last = (my_nblk - 1) % NS
        for b in range(NS):
            @pl.when(last == b)
            def _():
                wait_scatter(b)

    def drain(sum_hbm, cnt_hbm, hoff, rezero):
        my_ch0 = s * NCH

        def hcopy(i, sem):
            rr = (my_ch0 + i) * DR
            return pltpu.make_async_copy(
                ssum.at[pl.ds(rr, DR)],
                sum_hbm.at[pl.ds(rr, DR), pl.ds(hoff, H)], sem)

        for i in range(NS):
            hcopy(i, ssems[i]).start()
        hcopy(0, ssems[0]).wait()
        if rezero:
            pltpu.sync_copy(zbuf, ssum.at[pl.ds(my_ch0 * DR, DR)])
        hcopy(NS, ssems[0]).start()
        for i in range(1, NS):
            hcopy(i, ssems[i]).wait()
            if rezero:
                pltpu.sync_copy(zbuf, ssum.at[pl.ds((my_ch0 + i) * DR, DR)])
        hcopy(NS, ssems[0]).wait()
        if rezero:
            pltpu.sync_copy(zbuf, ssum.at[pl.ds((my_ch0 + NS) * DR, DR)])
        if cnt_hbm is not None:
            pltpu.sync_copy(cnt_part, cnt_hbm.at[s])

    @pl.when(c == 0)
    def _():
        side_loop(feat1_hbm, grid1_hbm, NB1, 0, True)
        side_start(feat2_hbm, grid2_hbm, NB2, 0)

    @pl.when(c == 1)
    def _():
        side_loop(feat1_hbm, grid1_hbm, NB1, H, False)
        side_start(feat2_hbm, grid2_hbm, NB2, H)

    plsc.subcore_barrier()

    @pl.when(c == 0)
    def _():
        drain(sum1_hbm, cnt1_hbm, 0, True)

    @pl.when(c == 1)
    def _():
        drain(sum1_hbm, None, H, True)

    plsc.subcore_barrier()

    @pl.when(c == 0)
    def _():
        side_loop(feat2_hbm, grid2_hbm, NB2, 0, False)

    @pl.when(c == 1)
    def _():
        side_loop(feat2_hbm, grid2_hbm, NB2, H, True)

    plsc.subcore_barrier()

    @pl.when(c == 0)
    def _():
        drain(sum2_hbm, None, 0, False)

    @pl.when(c == 1)
    def _():
        drain(sum2_hbm, cnt2_hbm, H, False)


@jax.jit
def _aggregate(feat1, grid1, feat2, grid2):
    mesh = plsc.VectorSubcoreMesh(core_axis_name="c", subcore_axis_name="s")
    f32 = jnp.float32
    z2 = jnp.zeros((ZR, H), f32)
    z1 = jnp.zeros((G,), f32)
    return pl.kernel(
        _sc_body,
        out_type=[
            jax.ShapeDtypeStruct((G, C), f32),
            jax.ShapeDtypeStruct((G, C), f32),
            jax.ShapeDtypeStruct((NTILES, G), f32),
            jax.ShapeDtypeStruct((NTILES, G), f32),
        ],
        mesh=mesh,
        compiler_params=pltpu.CompilerParams(use_tc_tiling_on_sc=False,
                                             needs_layout_passes=False),
        scratch_types=[
            pltpu.VMEM((BLK, H), f32),
            pltpu.VMEM((BLK, H), f32),
            pltpu.VMEM((BLK, H), f32),
            pltpu.VMEM((BLK, H), f32),
            pltpu.VMEM((BLK,), jnp.int32),
            pltpu.VMEM((BLK,), jnp.int32),
            pltpu.VMEM((BLK,), jnp.int32),
            pltpu.VMEM((BLK,), jnp.int32),
            pltpu.VMEM((G,), f32),
            pltpu.VMEM((ZR, H), f32),
            pltpu.SemaphoreType.DMA,
            pltpu.SemaphoreType.DMA,
            pltpu.SemaphoreType.DMA,
            pltpu.SemaphoreType.DMA,
            pltpu.SemaphoreType.DMA,
            pltpu.SemaphoreType.DMA,
            pltpu.SemaphoreType.DMA,
            pltpu.SemaphoreType.DMA,
            pltpu.SemaphoreType.DMA,
            pltpu.SemaphoreType.DMA,
            pltpu.SemaphoreType.DMA,
            pltpu.SemaphoreType.DMA,
            pltpu.VMEM_SHARED((G, H), f32),
        ],
    )(feat1, grid1, feat2, grid2, z2, z1)


def _loss_body(s1_ref, s2_ref, c1_ref, c2_ref, out_ref):
    cnt1 = jnp.sum(c1_ref[...], axis=0)[:, None]
    cnt2 = jnp.sum(c2_ref[...], axis=0)[:, None]
    mask = jnp.logical_and(cnt1 > 0.0, cnt2 > 0.0).astype(jnp.float32)
    d = s1_ref[...] / jnp.maximum(cnt1, 1.0) - s2_ref[...] / jnp.maximum(
        cnt2, 1.0)
    total = jnp.sum(d * d * mask)
    n = jnp.sum(mask)
    loss = jnp.where(n > 0.0, total / (n * jnp.float32(C)), jnp.float32(0.0))
    out_ref[...] = jnp.broadcast_to(loss, (1, 1))


@jax.jit
def kernel(feat1, grid1, feat2, grid2):
    grid1 = grid1.astype(jnp.int32)
    grid2 = grid2.astype(jnp.int32)
    sum1, sum2, cnt1, cnt2 = _aggregate(feat1, grid1, feat2, grid2)
    out = pl.pallas_call(
        _loss_body,
        out_shape=jax.ShapeDtypeStruct((1, 1), jnp.float32),
    )(sum1, sum2, cnt1, cnt2)
    return out[0, 0]


# speedup vs baseline: 10.0825x; 1.0183x over previous
import jax
import jax.numpy as jnp
from jax import lax
from jax.experimental import pallas as pl
from jax.experimental.pallas import tpu as pltpu
from jax.experimental.pallas import tpu_sc as plsc

N1 = 320000
N2 = 160000
C = 128
H = C // 2
G = 10000

BLK = 256
SCAT = 128
NB1 = N1 // BLK
NB2 = N2 // BLK
NTILES = 16
ZROWS = G // NTILES
ZR = 125
DR = 125
NDCH = G // DR
NCH = NDCH // NTILES


def _sc_body(feat1_hbm, grid1_hbm, feat2_hbm, grid2_hbm, z2_hbm, z1_hbm,
             sum1_hbm, sum2_hbm, cnt1_hbm, cnt2_hbm,
             half0, half1, half2, half3, idx0, idx1, idx2, idx3,
             cnt_part, zbuf,
             semf0, semf1, semf2, semf3, semi0, semi1, semi2, semi3,
             sems0, sems1, sems2, sems3, ssum):
    c = lax.axis_index("c")
    s = lax.axis_index("s")

    NS = 4
    halves = (half0, half1, half2, half3)
    idxs = (idx0, idx1, idx2, idx3)
    fsems = (semf0, semf1, semf2, semf3)
    isems = (semi0, semi1, semi2, semi3)
    ssems = (sems0, sems1, sems2, sems3)

    def my_block_range(nb):
        base_blk = nb // NTILES
        extra = nb % NTILES
        my_nblk = base_blk + jnp.where(s < extra, 1, 0)
        my_start = s * base_blk + jnp.minimum(s, extra)
        return my_start, my_nblk

    def load_block(feat_hbm, grid_hbm, hoff, my_start, j, slot):
        row0 = (my_start + j) * BLK
        pltpu.make_async_copy(grid_hbm.at[pl.ds(row0, BLK)],
                              idxs[slot], isems[slot]).start()
        pltpu.make_async_copy(
            feat_hbm.at[pl.ds(row0, BLK), pl.ds(hoff, H)],
            halves[slot], fsems[slot]).start()

    def side_start(feat_hbm, grid_hbm, nb, hoff):
        my_start, _ = my_block_range(nb)
        for j0 in range(NS - 1):
            load_block(feat_hbm, grid_hbm, hoff, my_start, j0, j0)

    @pl.when(c == 0)
    def _():
        side_start(feat1_hbm, grid1_hbm, NB1, 0)

    @pl.when(c == 1)
    def _():
        side_start(feat1_hbm, grid1_hbm, NB1, H)

    pltpu.sync_copy(z2_hbm, zbuf)
    pltpu.sync_copy(z1_hbm, cnt_part)

    r0 = s * ZROWS
    for k in range(ZROWS // ZR):
        pltpu.sync_copy(zbuf, ssum.at[pl.ds(r0 + k * ZR, ZR)])
    plsc.subcore_barrier()

    def side_loop(feat_hbm, grid_hbm, nb, hoff, with_cnt):
        my_start, my_nblk = my_block_range(nb)

        def wait_load(slot):
            pltpu.make_async_copy(grid_hbm.at[pl.ds(0, BLK)],
                                  idxs[slot], isems[slot]).wait()
            pltpu.make_async_copy(
                feat_hbm.at[pl.ds(0, BLK), pl.ds(0, H)],
                halves[slot], fsems[slot]).wait()

        def wait_scatter(slot):
            for p in range(BLK // SCAT):
                pltpu.make_async_copy(
                    halves[slot].at[pl.ds(p * SCAT, SCAT)],
                    ssum.at[idxs[slot].at[pl.ds(p * SCAT, SCAT)]],
                    ssems[slot]).wait()

        def quad(q, _):
            for b in range(NS):
                j = q * NS + b

                @pl.when(j < my_nblk)
                def _():
                    wait_load(b)
                    for p in range(BLK // SCAT):
                        pltpu.async_copy(
                            halves[b].at[pl.ds(p * SCAT, SCAT)],
                            ssum.at[idxs[b].at[pl.ds(p * SCAT, SCAT)]],
                            ssems[b], add=True)
                    if with_cnt:
                        ones16 = jnp.ones((16,), jnp.float32)
                        for k in range(BLK // 16):
                            idxv = idxs[b][pl.ds(k * 16, 16)]
                            plsc.addupdate_scatter(cnt_part, [idxv], ones16)

                    @pl.when(j >= 1)
                    def _():
                        wait_scatter((b + NS - 1) % NS)

                    @pl.when(j + NS - 1 < my_nblk)
                    def _():
                        load_block(feat_hbm, grid_hbm, hoff, my_start,
                                   j + NS - 1, (b + NS - 1) % NS)

            return 0

        lax.fori_loop(0, (my_nblk + NS - 1) // NS, quad, 0)

        last = (my_nblk - 1) % NS
        for b in range(NS):
            @pl.when(last == b)
            def _():
                wait_scatter(b)

    def drain(sum_hbm, cnt_hbm, hoff, rezero):
        my_ch0 = s * NCH

        def hcopy(i, sem):
            rr = (my_ch0 + i) * DR
            return pltpu.make_async_copy(
                ssum.at[pl.ds(rr, DR)],
                sum_hbm.at[pl.ds(rr, DR), pl.ds(hoff, H)], sem)

        for i in range(NS):
            hcopy(i, ssems[i]).start()
        hcopy(0, ssems[0]).wait()
        if rezero:
            pltpu.sync_copy(zbuf, ssum.at[pl.ds(my_ch0 * DR, DR)])
        hcopy(NS, ssems[0]).start()
        for i in range(1, NS):
            hcopy(i, ssems[i]).wait()
            if rezero:
                pltpu.sync_copy(zbuf, ssum.at[pl.ds((my_ch0 + i) * DR, DR)])
        hcopy(NS, ssems[0]).wait()
        if rezero:
            pltpu.sync_copy(zbuf, ssum.at[pl.ds((my_ch0 + NS) * DR, DR)])
        if cnt_hbm is not None:
            pltpu.sync_copy(cnt_part, cnt_hbm.at[s])

    @pl.when(c == 0)
    def _():
        side_loop(feat1_hbm, grid1_hbm, NB1, 0, True)
        side_start(feat2_hbm, grid2_hbm, NB2, 0)

    @pl.when(c == 1)
    def _():
        side_loop(feat1_hbm, grid1_hbm, NB1, H, False)
        side_start(feat2_hbm, grid2_hbm, NB2, H)

    plsc.subcore_barrier()

    @pl.when(c == 0)
    def _():
        drain(sum1_hbm, cnt1_hbm, 0, True)

    @pl.when(c == 1)
    def _():
        drain(sum1_hbm, None, H, True)

    plsc.subcore_barrier()

    @pl.when(c == 0)
    def _():
        side_loop(feat2_hbm, grid2_hbm, NB2, 0, False)

    @pl.when(c == 1)
    def _():
        side_loop(feat2_hbm, grid2_hbm, NB2, H, True)

    plsc.subcore_barrier()

    @pl.when(c == 0)
    def _():
        drain(sum2_hbm, None, 0, False)

    @pl.when(c == 1)
    def _():
        drain(sum2_hbm, cnt2_hbm, H, False)


@jax.jit
def _aggregate(feat1, grid1, feat2, grid2):
    mesh = plsc.VectorSubcoreMesh(core_axis_name="c", subcore_axis_name="s")
    f32 = jnp.float32
    z2 = jnp.zeros((ZR, H), f32)
    z1 = jnp.zeros((G,), f32)
    return pl.kernel(
        _sc_body,
        out_type=[
            jax.ShapeDtypeStruct((G, C), f32),
            jax.ShapeDtypeStruct((G, C), f32),
            jax.ShapeDtypeStruct((NTILES, G), f32),
            jax.ShapeDtypeStruct((NTILES, G), f32),
        ],
        mesh=mesh,
        compiler_params=pltpu.CompilerParams(use_tc_tiling_on_sc=False,
                                             needs_layout_passes=False),
        scratch_types=[
            pltpu.VMEM((BLK, H), f32),
            pltpu.VMEM((BLK, H), f32),
            pltpu.VMEM((BLK, H), f32),
            pltpu.VMEM((BLK, H), f32),
            pltpu.VMEM((BLK,), jnp.int32),
            pltpu.VMEM((BLK,), jnp.int32),
            pltpu.VMEM((BLK,), jnp.int32),
            pltpu.VMEM((BLK,), jnp.int32),
            pltpu.VMEM((G,), f32),
            pltpu.VMEM((ZR, H), f32),
            pltpu.SemaphoreType.DMA,
            pltpu.SemaphoreType.DMA,
            pltpu.SemaphoreType.DMA,
            pltpu.SemaphoreType.DMA,
            pltpu.SemaphoreType.DMA,
            pltpu.SemaphoreType.DMA,
            pltpu.SemaphoreType.DMA,
            pltpu.SemaphoreType.DMA,
            pltpu.SemaphoreType.DMA,
            pltpu.SemaphoreType.DMA,
            pltpu.SemaphoreType.DMA,
            pltpu.SemaphoreType.DMA,
            pltpu.VMEM_SHARED((G, H), f32),
        ],
    )(feat1, grid1, feat2, grid2, z2, z1)


def _loss_body(s1_ref, s2_ref, c1_ref, c2_ref, out_ref):
    cnt1 = jnp.sum(c1_ref[...], axis=0)[:, None]
    cnt2 = jnp.sum(c2_ref[...], axis=0)[:, None]
    mask = jnp.logical_and(cnt1 > 0.0, cnt2 > 0.0).astype(jnp.float32)
    d = s1_ref[...] / jnp.maximum(cnt1, 1.0) - s2_ref[...] / jnp.maximum(
        cnt2, 1.0)
    total = jnp.sum(d * d * mask)
    n = jnp.sum(mask)
    loss = jnp.where(n > 0.0, total / (n * jnp.float32(C)), jnp.float32(0.0))
    out_ref[...] = jnp.broadcast_to(loss, (1, 1))


@jax.jit
def kernel(feat1, grid1, feat2, grid2):
    grid1 = grid1.astype(jnp.int32)
    grid2 = grid2.astype(jnp.int32)
    sum1, sum2, cnt1, cnt2 = _aggregate(feat1, grid1, feat2, grid2)
    out = pl.pallas_call(
        _loss_body,
        out_shape=jax.ShapeDtypeStruct((1, 1), jnp.float32),
    )(sum1, sum2, cnt1, cnt2)
    return out[0, 0]
